# baseline (speedup 1.0000x reference)
import jax
import jax.numpy as jnp
from jax.experimental import pallas as pl
from jax.experimental.pallas import tpu as pltpu

_N, _F, _H = 10000, 128, 128
_BM = 400


def _sf_body(embs_ref, w_ref, sf_ref):
    sf_ref[...] = jnp.dot(
        embs_ref[...].astype(jnp.bfloat16),
        w_ref[...].astype(jnp.bfloat16),
        preferred_element_type=jnp.float32,
    ).astype(jnp.bfloat16)


def _gcn_body(sf_ref, adj_ref, b_ref, a_ref, out_ref):
    acc = jnp.dot(
        adj_ref[...].astype(jnp.bfloat16),
        sf_ref[...],
        preferred_element_type=jnp.float32,
    )
    o = acc + b_ref[...]
    a = a_ref[0, 0]
    out_ref[...] = jnp.where(o > 0, o, a * o)


def kernel(embs, adj, W, b, prelu_a):
    sf = pl.pallas_call(
        _sf_body,
        out_shape=jax.ShapeDtypeStruct((_N, _H), jnp.bfloat16),
    )(embs, W)
    return pl.pallas_call(
        _gcn_body,
        grid=(_N // _BM,),
        in_specs=[
            pl.BlockSpec((_N, _H), lambda m: (0, 0)),
            pl.BlockSpec((_BM, _N), lambda m: (m, 0)),
            pl.BlockSpec((1, _H), lambda m: (0, 0)),
            pl.BlockSpec((1, 1), lambda m: (0, 0), memory_space=pltpu.SMEM),
        ],
        out_specs=pl.BlockSpec((_BM, _H), lambda m: (m, 0)),
        out_shape=jax.ShapeDtypeStruct((_N, _H), jnp.float32),
        compiler_params=pltpu.CompilerParams(
            dimension_semantics=("parallel",),
        ),
    )(sf, adj, b.reshape(1, _H), prelu_a.reshape(1, 1))


# speedup vs baseline: 1.0353x; 1.0353x over previous
import jax
import jax.numpy as jnp
from jax.experimental import pallas as pl
from jax.experimental.pallas import tpu as pltpu

_N, _F, _H = 10000, 128, 128
_BM = 400


def _gcn_body(embs_ref, w_ref, adj_ref, b_ref, a_ref, out_ref, sf_ref):
    m = pl.program_id(0)

    @pl.when(m == 0)
    def _():
        sf_ref[...] = jnp.dot(
            embs_ref[...].astype(jnp.bfloat16),
            w_ref[...].astype(jnp.bfloat16),
            preferred_element_type=jnp.float32,
        ).astype(jnp.bfloat16)

    acc = jnp.dot(
        adj_ref[...].astype(jnp.bfloat16),
        sf_ref[...],
        preferred_element_type=jnp.float32,
    )
    o = acc + b_ref[...]
    a = a_ref[0, 0]
    out_ref[...] = jnp.where(o > 0, o, a * o)


def kernel(embs, adj, W, b, prelu_a):
    return pl.pallas_call(
        _gcn_body,
        grid=(_N // _BM,),
        in_specs=[
            pl.BlockSpec((_N, _F), lambda m: (0, 0)),
            pl.BlockSpec((_F, _H), lambda m: (0, 0)),
            pl.BlockSpec((_BM, _N), lambda m: (m, 0)),
            pl.BlockSpec((1, _H), lambda m: (0, 0)),
            pl.BlockSpec((1, 1), lambda m: (0, 0), memory_space=pltpu.SMEM),
        ],
        out_specs=pl.BlockSpec((_BM, _H), lambda m: (m, 0)),
        out_shape=jax.ShapeDtypeStruct((_N, _H), jnp.float32),
        scratch_shapes=[pltpu.VMEM((_N, _H), jnp.bfloat16)],
    )(embs, W, adj, b.reshape(1, _H), prelu_a.reshape(1, 1))
